# Initial kernel scaffold; baseline (speedup 1.0000x reference)
import jax
import jax.numpy as jnp
from jax import lax
from jax.experimental import pallas as pl
from jax.experimental.pallas import tpu as pltpu
from jax.experimental.pallas import tpu_sc as plsc

B, L = 4096, 200
N = B * L
VOCAB = 1001
D = 128
NC, NS, LANES = 2, 16, 16
NW = NC * NS
ROWS_PER_W = N // NW
CHUNK = 256
NCHUNK = ROWS_PER_W // CHUNK
NB = 2 * CHUNK // D


def _sc_body(codes_hbm, minutes_hbm, values_hbm, ctab_hbm, out_hbm,
             codes_v, minutes_v, vals_v, stage_v,
             cidx0, cidx1, cidx2, cidx3, sem):
    cidx = (cidx0, cidx1, cidx2, cidx3)
    wid = lax.axis_index("s") * NC + lax.axis_index("c")
    lane = lax.iota(jnp.int32, LANES)

    def chunk_body(g, _):
        base = wid * ROWS_PER_W + g * CHUNK
        pltpu.sync_copy(codes_hbm.at[pl.ds(base, CHUNK)], codes_v)
        pltpu.sync_copy(minutes_hbm.at[pl.ds(base, CHUNK)], minutes_v)
        pltpu.sync_copy(values_hbm.at[pl.ds(base, CHUNK)], vals_v)

        for i in range(CHUNK // LANES):
            c16 = codes_v[pl.ds(i * LANES, LANES)]
            m16 = minutes_v[pl.ds(i * LANES, LANES)] + VOCAB
            pos = 2 * lane + 32 * (i % (D // 32))
            plsc.store_scatter(cidx[i // (D // 32)], [pos], c16)
            plsc.store_scatter(cidx[i // (D // 32)], [pos + 1], m16)

        for j in range(NB):
            pltpu.async_copy(ctab_hbm.at[cidx[j]],
                             stage_v.at[pl.ds(j * D, D)], sem)
        for j in range(NB):
            pltpu.make_async_copy(ctab_hbm.at[cidx[j]],
                                  stage_v.at[pl.ds(j * D, D)], sem).wait()

        col = jnp.full((LANES,), D - 1, jnp.int32)
        for i in range(CHUNK // LANES):
            v16 = vals_v[pl.ds(i * LANES, LANES)]
            row = 2 * (i * LANES + lane)
            plsc.store_scatter(stage_v, [row, col], v16)

        pltpu.sync_copy(stage_v, out_hbm.at[pl.ds(2 * base, 2 * CHUNK)])
        return ()

    lax.fori_loop(0, NCHUNK, chunk_body, ())


@jax.jit
def kernel(codes, values, minutes, table, time_encoding):
    codes_f = codes.reshape(N)
    minutes_f = minutes.reshape(N)
    values_f = values.reshape(N)
    ptab = jnp.pad(table, ((0, 0), (0, 1)))
    ctab = jnp.concatenate([ptab, time_encoding], axis=0)

    mesh = plsc.VectorSubcoreMesh(core_axis_name="c", subcore_axis_name="s",
                                  num_cores=NC, num_subcores=NS)
    out2 = pl.kernel(
        _sc_body,
        out_type=jax.ShapeDtypeStruct((2 * N, D), jnp.float32),
        mesh=mesh,
        scratch_types=[
            pltpu.VMEM((CHUNK,), jnp.int32),
            pltpu.VMEM((CHUNK,), jnp.int32),
            pltpu.VMEM((CHUNK,), jnp.float32),
            pltpu.VMEM((2 * CHUNK, D), jnp.float32),
            pltpu.VMEM((D,), jnp.int32),
            pltpu.VMEM((D,), jnp.int32),
            pltpu.VMEM((D,), jnp.int32),
            pltpu.VMEM((D,), jnp.int32),
            pltpu.SemaphoreType.DMA,
        ],
    )(codes_f, minutes_f, values_f, ctab)
    return out2.reshape(B, L, 2 * D)


# speedup vs baseline: 2.9125x; 2.9125x over previous
import jax
import jax.numpy as jnp
from jax import lax
from jax.experimental import pallas as pl
from jax.experimental.pallas import tpu as pltpu
from jax.experimental.pallas import tpu_sc as plsc

B, L = 4096, 200
N = B * L
VOCAB = 1001
D = 128
NC, NS, LANES = 2, 16, 16
NW = NC * NS
ROWS_PER_W = N // NW
CHUNK = 256
NCHUNK = ROWS_PER_W // CHUNK
NB = 2 * CHUNK // D


def _sc_body(codes_hbm, minutes_hbm, values_hbm, ctab_hbm, out_hbm,
             codes_v, minutes_v, vals_v, stage_v,
             cidx0, cidx1, cidx2, cidx3, sem):
    cidx = (cidx0, cidx1, cidx2, cidx3)
    wid = lax.axis_index("s") * NC + lax.axis_index("c")
    lane = lax.iota(jnp.int32, LANES)

    def chunk_body(g, _):
        base = wid * ROWS_PER_W + g * CHUNK
        pltpu.sync_copy(codes_hbm.at[pl.ds(base, CHUNK)], codes_v)
        pltpu.sync_copy(minutes_hbm.at[pl.ds(base, CHUNK)], minutes_v)
        pltpu.sync_copy(values_hbm.at[pl.ds(base, CHUNK)], vals_v)

        for i in range(CHUNK // LANES):
            c16 = codes_v[pl.ds(i * LANES, LANES)]
            m16 = minutes_v[pl.ds(i * LANES, LANES)] + VOCAB
            pos = 2 * lane + 32 * (i % (D // 32))
            plsc.store_scatter(cidx[i // (D // 32)], [pos], c16)
            plsc.store_scatter(cidx[i // (D // 32)], [pos + 1], m16)

        for j in range(NB):
            pltpu.async_copy(ctab_hbm.at[cidx[j]],
                             stage_v.at[pl.ds(j * D, D)], sem)
        for j in range(NB):
            pltpu.make_async_copy(ctab_hbm.at[cidx[j]],
                                  stage_v.at[pl.ds(j * D, D)], sem).wait()

        col = jnp.full((LANES,), D - 1, jnp.int32)
        for i in range(CHUNK // LANES):
            v16 = vals_v[pl.ds(i * LANES, LANES)]
            row = 2 * (i * LANES + lane)
            plsc.store_scatter(stage_v, [row, col], v16)

        pltpu.sync_copy(stage_v, out_hbm.at[pl.ds(2 * base, 2 * CHUNK)])
        return ()

    lax.fori_loop(0, NCHUNK, chunk_body, ())


@jax.jit
def kernel(codes, values, minutes, table, time_encoding):
    codes_f = codes.reshape(N)
    minutes_f = minutes.reshape(N)
    values_f = values.reshape(N)
    ptab = jnp.pad(table, ((0, 0), (0, 1)))
    ctab = jnp.concatenate([ptab, time_encoding], axis=0)

    mesh = plsc.VectorSubcoreMesh(core_axis_name="c", subcore_axis_name="s",
                                  num_cores=NC, num_subcores=NS)
    out2 = pl.kernel(
        _sc_body,
        out_type=jax.ShapeDtypeStruct((2 * N, D), jnp.float32),
        mesh=mesh,
        compiler_params=pltpu.CompilerParams(needs_layout_passes=False),
        scratch_types=[
            pltpu.VMEM((CHUNK,), jnp.int32),
            pltpu.VMEM((CHUNK,), jnp.int32),
            pltpu.VMEM((CHUNK,), jnp.float32),
            pltpu.VMEM((2 * CHUNK, D), jnp.float32),
            pltpu.VMEM((D,), jnp.int32),
            pltpu.VMEM((D,), jnp.int32),
            pltpu.VMEM((D,), jnp.int32),
            pltpu.VMEM((D,), jnp.int32),
            pltpu.SemaphoreType.DMA,
        ],
    )(codes_f, minutes_f, values_f, ctab)
    return out2.reshape(B, L, 2 * D)


# speedup vs baseline: 5.3226x; 1.8275x over previous
import jax
import jax.numpy as jnp
from jax import lax
from jax.experimental import pallas as pl
from jax.experimental.pallas import tpu as pltpu
from jax.experimental.pallas import tpu_sc as plsc

B, L = 4096, 200
N = B * L
VOCAB = 1001
D = 128
TROWS = 1184
NC, NS, LANES = 2, 16, 16
NW = NC * NS
ROWS_PER_W = N // NW
CHUNK = 128
NCHUNK = ROWS_PER_W // CHUNK
NB = 2 * CHUNK // D
NPAIR = NCHUNK // 2


def _sc_body(codes_hbm, minutes_hbm, values_hbm, ctab_hbm, out_hbm,
             stab, codes_v, minutes_v, vals_v, stage_v, cidx, sems):
    wid = lax.axis_index("s") * NC + lax.axis_index("c")
    lane = lax.iota(jnp.int32, LANES)
    col127 = jnp.full((LANES,), D - 1, jnp.int32)
    isem, gsem, wsem = sems

    @pl.when(lax.axis_index("s") == 0)
    def _stage_table():
        pltpu.sync_copy(ctab_hbm, stab)
    plsc.subcore_barrier()

    def idx_copies(g, x):
        base = wid * ROWS_PER_W + g * CHUNK
        return (
            pltpu.make_async_copy(codes_hbm.at[pl.ds(base, CHUNK)],
                                  codes_v[x], isem[x]),
            pltpu.make_async_copy(minutes_hbm.at[pl.ds(base, CHUNK)],
                                  minutes_v[x], isem[x]),
            pltpu.make_async_copy(values_hbm.at[pl.ds(base, CHUNK)],
                                  vals_v[x], isem[x]),
        )

    def gather_copies(x):
        return tuple(
            pltpu.make_async_copy(stab.at[cidx[x][j]],
                                  stage_v[x].at[pl.ds(j * D, D)], gsem[x])
            for j in range(NB))

    def wb_copy(g, x):
        base = wid * ROWS_PER_W + g * CHUNK
        return pltpu.make_async_copy(
            stage_v[x], out_hbm.at[pl.ds(2 * base, 2 * CHUNK)], wsem[x])

    def run_chunk(p, x):
        g = 2 * p + x
        for c in idx_copies(g, x):
            c.wait()
        for i in range(CHUNK // LANES):
            c16 = codes_v[x][pl.ds(i * LANES, LANES)]
            m16 = minutes_v[x][pl.ds(i * LANES, LANES)] + VOCAB
            pos = 2 * lane + 32 * (i % (D // 32))
            plsc.store_scatter(cidx[x][i // (D // 32)], [pos], c16)
            plsc.store_scatter(cidx[x][i // (D // 32)], [pos + 1], m16)
        @pl.when(p >= 1)
        def _():
            wb_copy(g, x).wait()
        for c in gather_copies(x):
            c.start()
        if x == 0:
            for c in idx_copies(g + 1, 1):
                c.start()
        else:
            @pl.when(p <= NPAIR - 2)
            def _():
                for c in idx_copies(g + 1, 0):
                    c.start()
        for c in gather_copies(x):
            c.wait()
        for i in range(CHUNK // LANES):
            v16 = vals_v[x][pl.ds(i * LANES, LANES)]
            row = 2 * (i * LANES + lane)
            plsc.store_scatter(stage_v[x], [row, col127], v16)
        wb_copy(g, x).start()

    for c in idx_copies(0, 0):
        c.start()

    def pair_body(p, _):
        run_chunk(p, 0)
        run_chunk(p, 1)
        return ()

    lax.fori_loop(0, NPAIR, pair_body, ())
    wb_copy(0, 0).wait()
    wb_copy(0, 1).wait()


@jax.jit
def kernel(codes, values, minutes, table, time_encoding):
    codes_f = codes.reshape(N)
    minutes_f = minutes.reshape(N)
    values_f = values.reshape(N)
    ptab = jnp.pad(table, ((0, 0), (0, 1)))
    ctab = jnp.concatenate(
        [ptab, time_encoding,
         jnp.zeros((TROWS - VOCAB - 180, D), jnp.float32)], axis=0)

    mesh = plsc.VectorSubcoreMesh(core_axis_name="c", subcore_axis_name="s",
                                  num_cores=NC, num_subcores=NS)
    out2 = pl.kernel(
        _sc_body,
        out_type=jax.ShapeDtypeStruct((2 * N, D), jnp.float32),
        mesh=mesh,
        compiler_params=pltpu.CompilerParams(needs_layout_passes=False),
        scratch_types=[
            pltpu.VMEM_SHARED((TROWS, D), jnp.float32),
            [pltpu.VMEM((CHUNK,), jnp.int32)] * 2,
            [pltpu.VMEM((CHUNK,), jnp.int32)] * 2,
            [pltpu.VMEM((CHUNK,), jnp.float32)] * 2,
            [pltpu.VMEM((2 * CHUNK, D), jnp.float32)] * 2,
            [[pltpu.VMEM((D,), jnp.int32)] * NB] * 2,
            [[pltpu.SemaphoreType.DMA] * 2] * 3,
        ],
    )(codes_f, minutes_f, values_f, ctab)
    return out2.reshape(B, L, 2 * D)
